# Initial kernel scaffold; baseline (speedup 1.0000x reference)
import jax
import jax.numpy as jnp
from jax.experimental import pallas as pl


def kernel(t, h, W_e, b_e, W_n, b_n):
    raise NotImplementedError("write your pallas kernel here")


# speedup vs baseline: 57.0131x; 57.0131x over previous
import functools

import jax
import jax.numpy as jnp
from jax.experimental import pallas as pl
from jax.experimental.pallas import tpu as pltpu

P = 256
D = 16
ED = 16


def _gin_kernel(nodes_ref, nodesT_ref, We1_ref, We2T_ref, wd_ref, be_ref,
                Wn1T_ref, Wn2T_ref, bnc_ref, out_ref, agg_scr):
    nodes = nodes_ref[0]
    nT = nodesT_ref[0]

    g = jax.lax.dot_general(nT, nT, (((0,), (0,)), ((), ())),
                            preferred_element_type=jnp.float32)
    sq_row = jnp.sum(nT * nT, axis=0, keepdims=True)
    sq_col = jnp.sum(nodes * nodes, axis=1, keepdims=True)
    dist = jnp.sqrt(jnp.maximum(sq_col + sq_row - 2.0 * g, 0.0))

    a2 = jax.lax.dot_general(nodes, We1_ref[...], (((1,), (0,)), ((), ())),
                             preferred_element_type=jnp.float32) + be_ref[...]
    cT = jax.lax.dot_general(We2T_ref[...], nT, (((1,), (0,)), ((), ())),
                             preferred_element_type=jnp.float32)

    rows = jax.lax.broadcasted_iota(jnp.int32, (P, P), 0)
    cols = jax.lax.broadcasted_iota(jnp.int32, (P, P), 1)
    mask = (rows != cols).astype(jnp.float32)

    for k in range(ED):
        m = dist * wd_ref[0, k] + a2[:, k:k + 1] + cT[k:k + 1, :]
        m = jnp.maximum(m, 0.0) * mask
        agg_scr[k:k + 1, :] = jnp.sum(m, axis=0, keepdims=True)

    aggT = agg_scr[...]
    newT = (jax.lax.dot_general(Wn1T_ref[...], aggT, (((1,), (0,)), ((), ())),
                                preferred_element_type=jnp.float32)
            + jax.lax.dot_general(Wn2T_ref[...], nT, (((1,), (0,)), ((), ())),
                                  preferred_element_type=jnp.float32)
            + bnc_ref[...])
    out_ref[0] = newT


@functools.partial(jax.jit, static_argnames=())
def kernel(t, h, W_e, b_e, W_n, b_n):
    del t
    B = h.shape[0]
    nodes = h.reshape(B, P, D)
    nodesT = nodes.transpose(0, 2, 1)

    We1 = W_e[:D]
    We2T = W_e[D:2 * D].T
    wd = W_e[2 * D:2 * D + 1]
    be = b_e.reshape(1, ED)
    Wn1T = W_n[:ED].T
    Wn2T = W_n[ED:].T
    bnc = b_n.reshape(D, 1)

    full = lambda shape: pl.BlockSpec(shape, lambda b: (0,) * len(shape))
    outT = pl.pallas_call(
        _gin_kernel,
        grid=(B,),
        in_specs=[
            pl.BlockSpec((1, P, D), lambda b: (b, 0, 0)),
            pl.BlockSpec((1, D, P), lambda b: (b, 0, 0)),
            full((D, ED)), full((ED, D)), full((1, ED)), full((1, ED)),
            full((D, ED)), full((D, D)), full((D, 1)),
        ],
        out_specs=pl.BlockSpec((1, D, P), lambda b: (b, 0, 0)),
        out_shape=jax.ShapeDtypeStruct((B, D, P), jnp.float32),
        scratch_shapes=[pltpu.VMEM((ED, P), jnp.float32)],
    )(nodes, nodesT, We1, We2T, wd, be, Wn1T, Wn2T, bnc)

    return outT.transpose(0, 2, 1).reshape(B, P * D)
